# baseline (speedup 1.0000x reference)
import functools

import jax
import jax.numpy as jnp
from jax import lax
from jax.experimental import pallas as pl
from jax.experimental.pallas import tpu as pltpu
from jax.experimental.pallas import tpu_sc as plsc

_B = 4096
_K = 200
_D = 128
_COORD = 16
_U = 1000
_F = _D + _COORD + 3

_NC = 2
_NS = 16
_NW = _NC * _NS
_BC = _B // _NW
_NCH = _BC // 16


def _sc_gather_kernel(unc_hbm, act_hbm, out_hbm, act_v, idx_v, row_v, sem):
    cid = lax.axis_index("c")
    sid = lax.axis_index("s")
    wid = sid * _NC + cid
    b0 = wid * _BC

    pltpu.sync_copy(act_hbm.at[:, pl.ds(b0, _BC)], act_v)

    def k_body(k, carry):
        for ch in range(_NCH):
            a = act_v[k, pl.ds(ch * 16, 16)]
            lo = jnp.zeros((16,), jnp.int32)
            hi = jnp.full((16,), _U - 1, jnp.int32)
            ac = jnp.minimum(jnp.maximum(a, lo), hi)
            three = jnp.full((16,), 3, jnp.int32)
            seven = jnp.full((16,), 7, jnp.int32)
            g = (lax.shift_right_logical(ac, three)
                 * jnp.full((16,), (_B // 128) * 1024, jnp.int32)
                 + jnp.bitwise_and(ac, seven) * jnp.full((16,), 128, jnp.int32)
                 + lax.broadcast(wid * 1024 + ch * 16, (16,))
                 + lax.iota(jnp.int32, 16))
            idx_v[pl.ds(k * _BC + ch * 16, 16)] = g
        return carry

    lax.fori_loop(0, _K, k_body, 0)
    pltpu.async_copy(unc_hbm.at[idx_v], row_v, sem).wait()

    def fire_body(k, carry):
        pltpu.make_async_copy(
            row_v.at[pl.ds(k * _BC, _BC)],
            out_hbm.at[k, pl.ds(b0, _BC)], sem).start()
        return carry

    def drain_body(k, carry):
        pltpu.make_async_copy(
            row_v.at[pl.ds(k * _BC, _BC)],
            out_hbm.at[k, pl.ds(b0, _BC)], sem).wait()
        return carry

    lax.fori_loop(0, _K, fire_body, 0)
    lax.fori_loop(0, _K, drain_body, 0)


def _sc_gather_t(uncertainty, actions):
    unc_flat_t = (uncertainty.T.reshape(_U // 8, 8, _B // 128, 128)
                  .transpose(0, 2, 1, 3).reshape(-1))
    act_t = actions.astype(jnp.int32).T
    mesh = plsc.VectorSubcoreMesh(core_axis_name="c", subcore_axis_name="s")
    k = functools.partial(
        pl.kernel,
        mesh=mesh,
        out_type=jax.ShapeDtypeStruct((_K, _B), jnp.float32),
        scratch_types=[
            pltpu.VMEM((_K, _BC), jnp.int32),
            pltpu.VMEM((_K * _BC,), jnp.int32),
            pltpu.VMEM((_K * _BC,), jnp.float32),
            pltpu.SemaphoreType.DMA,
        ],
        compiler_params=pltpu.CompilerParams(needs_layout_passes=False),
    )(_sc_gather_kernel)
    return k(unc_flat_t, act_t)


_BBLK = 128
_KBLK = 40


def _tc_assemble_kernel(f_ref, c_ref, s_ref, b_ref, o_ref):
    ft = jnp.transpose(f_ref[...], (1, 0))
    o_ref[0:_D, :, :] = jnp.broadcast_to(
        ft[:, None, :], (_D, _KBLK, _BBLK))
    cv = c_ref[...]
    for c in range(_COORD):
        o_ref[pl.ds(_D + c, 1), :, :] = cv[:, c, :][None]
    o_ref[pl.ds(_D + _COORD, 1), :, :] = s_ref[...][None]
    o_ref[pl.ds(_D + _COORD + 2, 1), :, :] = jnp.broadcast_to(
        b_ref[...], (_KBLK, _BBLK))[None]


def _tc_assemble(features, coords, centered_scores, boundary_risk):
    coords_t = jnp.transpose(coords, (1, 2, 0))
    cs_t = jnp.transpose(centered_scores, (1, 0))
    br2 = boundary_risk.reshape(1, _B)
    return pl.pallas_call(
        _tc_assemble_kernel,
        grid=(_B // _BBLK, _K // _KBLK),
        in_specs=[
            pl.BlockSpec((_BBLK, _D), lambda i, j: (i, 0)),
            pl.BlockSpec((_KBLK, _COORD, _BBLK), lambda i, j: (j, 0, i)),
            pl.BlockSpec((_KBLK, _BBLK), lambda i, j: (j, i)),
            pl.BlockSpec((1, _BBLK), lambda i, j: (0, i)),
        ],
        out_specs=pl.BlockSpec((_F, _KBLK, _BBLK), lambda i, j: (0, j, i)),
        out_shape=jax.ShapeDtypeStruct((_F, _K, _B), jnp.float32),
    )(features, coords_t, cs_t, br2)


def _tc_patch_kernel(t_ref, u_ref, o_ref):
    o_ref[...] = u_ref[...]


def _tc_patch_unc(t, unc_t):
    return pl.pallas_call(
        _tc_patch_kernel,
        grid=(1,),
        in_specs=[
            pl.BlockSpec(memory_space=pl.ANY),
            pl.BlockSpec((1, _K, _B), lambda i: (0, 0, 0)),
        ],
        out_specs=pl.BlockSpec((1, _K, _B), lambda i: (_D + _COORD + 1, 0, 0)),
        out_shape=jax.ShapeDtypeStruct((_F, _K, _B), jnp.float32),
        input_output_aliases={0: 0},
    )(t, unc_t.reshape(1, _K, _B))


def kernel(features, coords, actions, uncertainty, centered_scores, boundary_risk):
    if features.ndim > 2:
        features = features.reshape(features.shape[0], -1)
    unc_t = _sc_gather_t(uncertainty, actions)
    t = _tc_assemble(features, coords.astype(jnp.float32),
                     centered_scores, boundary_risk)
    t = _tc_patch_unc(t, unc_t)
    feats = jnp.transpose(t, (2, 1, 0))
    action_mask = jnp.ones((_B, _K), dtype=bool)
    return (feats, action_mask)


# speedup vs baseline: 1.1946x; 1.1946x over previous
import functools

import jax
import jax.numpy as jnp
from jax import lax
from jax.experimental import pallas as pl
from jax.experimental.pallas import tpu as pltpu
from jax.experimental.pallas import tpu_sc as plsc

_B = 4096
_K = 200
_D = 128
_COORD = 16
_U = 1000
_F = _D + _COORD + 3

_NC = 2
_NS = 16
_NW = _NC * _NS
_BC = _B // _NW
_NCH = _BC // 16


def _sc_gather_kernel(unc_hbm, act_hbm, out_hbm, act_v, idx_v, row_v, sem):
    cid = lax.axis_index("c")
    sid = lax.axis_index("s")
    wid = sid * _NC + cid
    b0 = wid * _BC

    pltpu.sync_copy(act_hbm.at[:, pl.ds(b0, _BC)], act_v)

    def k_body(k, carry):
        for ch in range(_NCH):
            a = act_v[k, pl.ds(ch * 16, 16)]
            lo = jnp.zeros((16,), jnp.int32)
            hi = jnp.full((16,), _U - 1, jnp.int32)
            ac = jnp.minimum(jnp.maximum(a, lo), hi)
            three = jnp.full((16,), 3, jnp.int32)
            seven = jnp.full((16,), 7, jnp.int32)
            g = (lax.shift_right_logical(ac, three)
                 * jnp.full((16,), (_B // 128) * 1024, jnp.int32)
                 + jnp.bitwise_and(ac, seven) * jnp.full((16,), 128, jnp.int32)
                 + lax.broadcast(wid * 1024 + ch * 16, (16,))
                 + lax.iota(jnp.int32, 16))
            idx_v[pl.ds(k * _BC + ch * 16, 16)] = g
        return carry

    lax.fori_loop(0, _K, k_body, 0)
    pltpu.async_copy(unc_hbm.at[idx_v], row_v, sem).wait()

    def fire_body(k, carry):
        pltpu.make_async_copy(
            row_v.at[pl.ds(k * _BC, _BC)],
            out_hbm.at[k, pl.ds(b0, _BC)], sem).start()
        return carry

    def drain_body(k, carry):
        pltpu.make_async_copy(
            row_v.at[pl.ds(k * _BC, _BC)],
            out_hbm.at[k, pl.ds(b0, _BC)], sem).wait()
        return carry

    lax.fori_loop(0, _K, fire_body, 0)
    lax.fori_loop(0, _K, drain_body, 0)


def _sc_gather_t(uncertainty, actions):
    unc_flat_t = (uncertainty.T.reshape(_U // 8, 8, _B // 128, 128)
                  .transpose(0, 2, 1, 3).reshape(-1))
    act_t = actions.astype(jnp.int32).T
    mesh = plsc.VectorSubcoreMesh(core_axis_name="c", subcore_axis_name="s")
    k = functools.partial(
        pl.kernel,
        mesh=mesh,
        out_type=jax.ShapeDtypeStruct((_K, _B), jnp.float32),
        scratch_types=[
            pltpu.VMEM((_K, _BC), jnp.int32),
            pltpu.VMEM((_K * _BC,), jnp.int32),
            pltpu.VMEM((_K * _BC,), jnp.float32),
            pltpu.SemaphoreType.DMA,
        ],
        compiler_params=pltpu.CompilerParams(needs_layout_passes=False),
    )(_sc_gather_kernel)
    return k(unc_flat_t, act_t)


_BBLK = 128
_KBLK = 200


def _tc_assemble_kernel(f_ref, c_ref, s_ref, b_ref, o_ref):
    ft = jnp.transpose(f_ref[...], (1, 0))
    o_ref[0:_D, :, :] = jnp.broadcast_to(
        ft[:, None, :], (_D, _KBLK, _BBLK))
    cv = c_ref[...]
    for c in range(_COORD):
        o_ref[pl.ds(_D + c, 1), :, :] = cv[:, c, :][None]
    o_ref[pl.ds(_D + _COORD, 1), :, :] = s_ref[...][None]
    o_ref[pl.ds(_D + _COORD + 2, 1), :, :] = jnp.broadcast_to(
        b_ref[...], (_KBLK, _BBLK))[None]


def _tc_assemble(features, coords, centered_scores, boundary_risk):
    coords_t = jnp.transpose(coords, (1, 2, 0))
    cs_t = jnp.transpose(centered_scores, (1, 0))
    br2 = boundary_risk.reshape(1, _B)
    return pl.pallas_call(
        _tc_assemble_kernel,
        grid=(_B // _BBLK, _K // _KBLK),
        in_specs=[
            pl.BlockSpec((_BBLK, _D), lambda i, j: (i, 0)),
            pl.BlockSpec((_KBLK, _COORD, _BBLK), lambda i, j: (j, 0, i)),
            pl.BlockSpec((_KBLK, _BBLK), lambda i, j: (j, i)),
            pl.BlockSpec((1, _BBLK), lambda i, j: (0, i)),
        ],
        out_specs=pl.BlockSpec((_F, _KBLK, _BBLK), lambda i, j: (0, j, i)),
        out_shape=jax.ShapeDtypeStruct((_F, _K, _B), jnp.float32),
    )(features, coords_t, cs_t, br2)


def _tc_patch_kernel(t_ref, u_ref, o_ref):
    o_ref[...] = u_ref[...]


def _tc_patch_unc(t, unc_t):
    return pl.pallas_call(
        _tc_patch_kernel,
        grid=(1,),
        in_specs=[
            pl.BlockSpec(memory_space=pl.ANY),
            pl.BlockSpec((1, _K, _B), lambda i: (0, 0, 0)),
        ],
        out_specs=pl.BlockSpec((1, _K, _B), lambda i: (_D + _COORD + 1, 0, 0)),
        out_shape=jax.ShapeDtypeStruct((_F, _K, _B), jnp.float32),
        input_output_aliases={0: 0},
    )(t, unc_t.reshape(1, _K, _B))


def kernel(features, coords, actions, uncertainty, centered_scores, boundary_risk):
    if features.ndim > 2:
        features = features.reshape(features.shape[0], -1)
    unc_t = _sc_gather_t(uncertainty, actions)
    t = _tc_assemble(features, coords.astype(jnp.float32),
                     centered_scores, boundary_risk)
    t = _tc_patch_unc(t, unc_t)
    feats = jnp.transpose(t, (2, 1, 0))
    action_mask = jnp.ones((_B, _K), dtype=bool)
    return (feats, action_mask)
